# baseline (speedup 1.0000x reference)
import functools

import jax
import jax.numpy as jnp
from jax import lax
from jax.experimental import pallas as pl
from jax.experimental.pallas import tpu as pltpu
from jax.experimental.pallas import tpu_sc as plsc

_CHUNK = 400


def kernel(emb_h, h, e, conv_w, conv_b, w2, b2, edge_index):
    E, H = e.shape
    info = plsc.get_sparse_core_info()
    nc, ns = info.num_cores, info.num_subcores
    nw = nc * ns
    rows_pw = E // nw
    n_iter = rows_pw // _CHUNK

    mesh = plsc.VectorSubcoreMesh(core_axis_name="c", subcore_axis_name="s")

    @functools.partial(
        pl.kernel,
        out_type=jax.ShapeDtypeStruct((E, H), e.dtype),
        mesh=mesh,
        scratch_types=[pltpu.VMEM((_CHUNK, H), e.dtype)],
    )
    def sc_copy(e_hbm, out_hbm, buf):
        wid = lax.axis_index("s") * nc + lax.axis_index("c")
        base = wid * rows_pw

        @pl.loop(0, n_iter)
        def _body(j):
            off = base + j * _CHUNK
            pltpu.sync_copy(e_hbm.at[pl.ds(off, _CHUNK)], buf)
            pltpu.sync_copy(buf, out_hbm.at[pl.ds(off, _CHUNK)])

    return sc_copy(e)


# speedup vs baseline: 1.0572x; 1.0572x over previous
import functools

import jax
import jax.numpy as jnp
from jax import lax
from jax.experimental import pallas as pl
from jax.experimental.pallas import tpu as pltpu
from jax.experimental.pallas import tpu_sc as plsc

_CHUNK = 200
_NBUF = 2


def kernel(emb_h, h, e, conv_w, conv_b, w2, b2, edge_index):
    E, H = e.shape
    info = plsc.get_sparse_core_info()
    nc, ns = info.num_cores, info.num_subcores
    nw = nc * ns
    rows_pw = E // nw
    n_iter = rows_pw // _CHUNK
    nsteady = n_iter - _NBUF

    mesh = plsc.VectorSubcoreMesh(core_axis_name="c", subcore_axis_name="s")

    @functools.partial(
        pl.kernel,
        out_type=jax.ShapeDtypeStruct((E, H), e.dtype),
        mesh=mesh,
        scratch_types=(
            [pltpu.VMEM((_CHUNK, H), e.dtype)] * _NBUF
            + [pltpu.SemaphoreType.DMA] * _NBUF
        ),
    )
    def sc_copy(e_hbm, out_hbm, b0, b1, s0, s1):
        bufs = (b0, b1)
        sems = (s0, s1)
        wid = lax.axis_index("s") * nc + lax.axis_index("c")
        base = wid * rows_pw
        C = _CHUNK

        for k in range(_NBUF):
            pltpu.async_copy(e_hbm.at[pl.ds(base + k * C, C)], bufs[k], sems[k])

        @pl.loop(0, nsteady, step=_NBUF)
        def _body(j):
            for k in range(_NBUF):
                off = base + (j + k) * C
                pltpu.make_async_copy(
                    e_hbm.at[pl.ds(off, C)], bufs[k], sems[k]).wait()
                pltpu.async_copy(bufs[k], out_hbm.at[pl.ds(off, C)], sems[k])
            for k in range(_NBUF):
                off = base + (j + k) * C
                noff = base + (j + _NBUF + k) * C
                pltpu.make_async_copy(
                    bufs[k], out_hbm.at[pl.ds(off, C)], sems[k]).wait()
                pltpu.async_copy(e_hbm.at[pl.ds(noff, C)], bufs[k], sems[k])

        for k in range(_NBUF):
            off = base + (nsteady + k) * C
            pltpu.make_async_copy(
                e_hbm.at[pl.ds(off, C)], bufs[k], sems[k]).wait()
            pltpu.async_copy(bufs[k], out_hbm.at[pl.ds(off, C)], sems[k])
        for k in range(_NBUF):
            off = base + (nsteady + k) * C
            pltpu.make_async_copy(
                bufs[k], out_hbm.at[pl.ds(off, C)], sems[k]).wait()

    return sc_copy(e)


# speedup vs baseline: 1.4346x; 1.3569x over previous
import jax
import jax.numpy as jnp
from jax.experimental import pallas as pl
from jax.experimental.pallas import tpu as pltpu

_BLOCK_ROWS = 10000


def _copy_body(e_ref, o_ref):
    o_ref[...] = e_ref[...]


def kernel(emb_h, h, e, conv_w, conv_b, w2, b2, edge_index):
    E, H = e.shape
    block_rows = _BLOCK_ROWS if E % _BLOCK_ROWS == 0 else E
    grid = (E // block_rows,)
    out = pl.pallas_call(
        _copy_body,
        grid=grid,
        in_specs=[pl.BlockSpec((block_rows, H), lambda i: (i, 0))],
        out_specs=pl.BlockSpec((block_rows, H), lambda i: (i, 0)),
        out_shape=jax.ShapeDtypeStruct((E, H), e.dtype),
        compiler_params=pltpu.CompilerParams(
            dimension_semantics=("parallel",),
        ),
    )(e)
    return out


# speedup vs baseline: 1.4490x; 1.0100x over previous
import jax
import jax.numpy as jnp
from jax.experimental import pallas as pl
from jax.experimental.pallas import tpu as pltpu

_BLOCK_ROWS = 20000


def _copy_body(e_ref, o_ref):
    o_ref[...] = e_ref[...]


def kernel(emb_h, h, e, conv_w, conv_b, w2, b2, edge_index):
    E, H = e.shape
    block_rows = _BLOCK_ROWS if E % _BLOCK_ROWS == 0 else E
    grid = (E // block_rows,)
    out = pl.pallas_call(
        _copy_body,
        grid=grid,
        in_specs=[pl.BlockSpec((block_rows, H), lambda i: (i, 0))],
        out_specs=pl.BlockSpec((block_rows, H), lambda i: (i, 0)),
        out_shape=jax.ShapeDtypeStruct((E, H), e.dtype),
    )(e)
    return out
